# Initial kernel scaffold; baseline (speedup 1.0000x reference)
import jax
import jax.numpy as jnp
from jax import lax
from jax.experimental import pallas as pl
from jax.experimental.pallas import tpu as pltpu
from jax.experimental.pallas import tpu_sc as plsc

N_TOK = 32768
N_EXP = 64
N_GRP = 8
GRP_SZ = 8
TOPK_GRP = 4
TOPK = 8
SCALE = 2.5

NC = 2
NS = 16
NW = NC * NS
TPW = N_TOK // NW
L = 16
TILES = TPW // L


def _i32(v):
    return jnp.full((L,), v, dtype=jnp.int32)


def _merge_top2(m1, s1, m2, s2):
    return (
        jnp.maximum(m1, m2),
        jnp.maximum(jnp.minimum(m1, m2), jnp.maximum(s1, s2)),
    )


def _tec_body(logits_hbm, bias_hbm, oi_hbm, ow_hbm, xs, s_buf, bias_v, oi_v, ow_v):
    wid = lax.axis_index("s") * NC + lax.axis_index("c")
    base = wid * TPW
    pltpu.sync_copy(logits_hbm.at[pl.ds(base, TPW), :], xs)
    pltpu.sync_copy(bias_hbm, bias_v)

    lanes = lax.iota(jnp.int32, L)
    neg_inf = jnp.full((L,), -jnp.inf, dtype=jnp.float32)
    zero = jnp.zeros((L,), dtype=jnp.float32)

    def tile(t, carry):
        tok_vec = t * L + lanes

        s = []
        sf = []
        for e in range(N_EXP):
            xe = plsc.load_gather(xs, [tok_vec, _i32(e)])
            se = 1.0 / (1.0 + jnp.exp(-xe))
            s_buf[e, :] = se
            s.append(se)
            sf.append(se + bias_v[e, :])

        gs = []
        for g in range(N_GRP):
            v = sf[GRP_SZ * g : GRP_SZ * (g + 1)]
            pm = [jnp.maximum(v[2 * i], v[2 * i + 1]) for i in range(4)]
            ps = [jnp.minimum(v[2 * i], v[2 * i + 1]) for i in range(4)]
            m01, s01 = _merge_top2(pm[0], ps[0], pm[1], ps[1])
            m23, s23 = _merge_top2(pm[2], ps[2], pm[3], ps[3])
            m, sec = _merge_top2(m01, s01, m23, s23)
            gs.append(m + sec)

        gsel = []
        for g in range(N_GRP):
            r = jnp.zeros((L,), dtype=jnp.int32)
            for h in range(N_GRP):
                if h == g:
                    continue
                c = (gs[h] >= gs[g]) if h < g else (gs[h] > gs[g])
                r = r + c.astype(jnp.int32)
            gsel.append(r < TOPK_GRP)

        sfc = [jnp.where(gsel[e // GRP_SZ], sf[e], zero) for e in range(N_EXP)]

        ws = []
        for k in range(TOPK):
            m = sfc[0]
            for e in range(1, N_EXP):
                m = jnp.maximum(m, sfc[e])
            idx = _i32(N_EXP)
            for e in range(N_EXP):
                idx = jnp.minimum(idx, jnp.where(sfc[e] == m, _i32(e), _i32(N_EXP)))
            plsc.store_scatter(oi_v, [tok_vec, _i32(k)], idx)
            ws.append(plsc.load_gather(s_buf, [idx, lanes]))
            sfc = [
                jnp.where(_i32(e) == idx, neg_inf, sfc[e]) for e in range(N_EXP)
            ]

        den = ws[0]
        for k in range(1, TOPK):
            den = den + ws[k]
        inv = SCALE / (den + 1e-20)
        for k in range(TOPK):
            plsc.store_scatter(ow_v, [tok_vec, _i32(k)], ws[k] * inv)
        return carry

    lax.fori_loop(0, TILES, tile, 0)
    pltpu.sync_copy(oi_v, oi_hbm.at[pl.ds(base, TPW), :])
    pltpu.sync_copy(ow_v, ow_hbm.at[pl.ds(base, TPW), :])


@jax.jit
def kernel(router_logits, e_score_correction_bias):
    bias_b = jnp.broadcast_to(
        e_score_correction_bias[:, None], (N_EXP, L)
    ).astype(jnp.float32)
    mesh = plsc.VectorSubcoreMesh(
        core_axis_name="c", subcore_axis_name="s", num_cores=NC, num_subcores=NS
    )
    f = pl.kernel(
        _tec_body,
        out_type=(
            jax.ShapeDtypeStruct((N_TOK, TOPK), jnp.int32),
            jax.ShapeDtypeStruct((N_TOK, TOPK), jnp.float32),
        ),
        mesh=mesh,
        scratch_types=[
            pltpu.VMEM((TPW, N_EXP), jnp.float32),
            pltpu.VMEM((N_EXP, L), jnp.float32),
            pltpu.VMEM((N_EXP, L), jnp.float32),
            pltpu.VMEM((TPW, TOPK), jnp.int32),
            pltpu.VMEM((TPW, TOPK), jnp.float32),
        ],
    )
    return f(router_logits, bias_b)


# speedup vs baseline: 2.3205x; 2.3205x over previous
import jax
import jax.numpy as jnp
from jax import lax
from jax.experimental import pallas as pl
from jax.experimental.pallas import tpu as pltpu
from jax.experimental.pallas import tpu_sc as plsc

N_TOK = 32768
N_EXP = 64
N_GRP = 8
GRP_SZ = 8
TOPK_GRP = 4
TOPK = 8
SCALE = 2.5

NC = 2
NS = 16
NW = NC * NS
TPW = N_TOK // NW
L = 16
TILES = TPW // L


def _i32(v):
    return jnp.full((L,), v, dtype=jnp.int32)


def _merge_top2(m1, s1, m2, s2):
    return (
        jnp.maximum(m1, m2),
        jnp.maximum(jnp.minimum(m1, m2), jnp.maximum(s1, s2)),
    )


def _tec_body(logits_hbm, bias_hbm, oi_hbm, ow_hbm, xs, s_buf, bias_v, oi_v, ow_v):
    wid = lax.axis_index("s") * NC + lax.axis_index("c")
    base = wid * TPW
    pltpu.sync_copy(logits_hbm.at[pl.ds(base * N_EXP, TPW * N_EXP)], xs)
    pltpu.sync_copy(bias_hbm, bias_v)

    lanes = lax.iota(jnp.int32, L)
    neg_inf = jnp.full((L,), -jnp.inf, dtype=jnp.float32)
    zero = jnp.zeros((L,), dtype=jnp.float32)

    def tile(t, carry):
        tok_vec = t * L + lanes
        xbase = tok_vec * N_EXP

        s = []
        sf = []
        for e in range(N_EXP):
            xe = plsc.load_gather(xs, [xbase + e])
            se = 1.0 / (1.0 + jnp.exp(-xe))
            s_buf[pl.ds(e * L, L)] = se
            s.append(se)
            sf.append(se + bias_v[pl.ds(e * L, L)])

        gs = []
        for g in range(N_GRP):
            v = sf[GRP_SZ * g : GRP_SZ * (g + 1)]
            pm = [jnp.maximum(v[2 * i], v[2 * i + 1]) for i in range(4)]
            ps = [jnp.minimum(v[2 * i], v[2 * i + 1]) for i in range(4)]
            m01, s01 = _merge_top2(pm[0], ps[0], pm[1], ps[1])
            m23, s23 = _merge_top2(pm[2], ps[2], pm[3], ps[3])
            m, sec = _merge_top2(m01, s01, m23, s23)
            gs.append(m + sec)

        gsel = []
        for g in range(N_GRP):
            r = jnp.zeros((L,), dtype=jnp.int32)
            for h in range(N_GRP):
                if h == g:
                    continue
                c = (gs[h] >= gs[g]) if h < g else (gs[h] > gs[g])
                r = r + c.astype(jnp.int32)
            gsel.append(r < TOPK_GRP)

        sfc = [jnp.where(gsel[e // GRP_SZ], sf[e], zero) for e in range(N_EXP)]

        obase = tok_vec * TOPK
        ws = []
        for k in range(TOPK):
            m = sfc[0]
            for e in range(1, N_EXP):
                m = jnp.maximum(m, sfc[e])
            idx = _i32(N_EXP)
            for e in range(N_EXP):
                idx = jnp.minimum(idx, jnp.where(sfc[e] == m, _i32(e), _i32(N_EXP)))
            plsc.store_scatter(oi_v, [obase + k], idx)
            ws.append(plsc.load_gather(s_buf, [idx * L + lanes]))
            sfc = [
                jnp.where(_i32(e) == idx, neg_inf, sfc[e]) for e in range(N_EXP)
            ]

        den = ws[0]
        for k in range(1, TOPK):
            den = den + ws[k]
        inv = SCALE / (den + 1e-20)
        for k in range(TOPK):
            plsc.store_scatter(ow_v, [obase + k], ws[k] * inv)
        return carry

    lax.fori_loop(0, TILES, tile, 0)
    pltpu.sync_copy(oi_v, oi_hbm.at[pl.ds(base * TOPK, TPW * TOPK)])
    pltpu.sync_copy(ow_v, ow_hbm.at[pl.ds(base * TOPK, TPW * TOPK)])


@jax.jit
def kernel(router_logits, e_score_correction_bias):
    logits_flat = router_logits.reshape(N_TOK * N_EXP)
    bias_b = jnp.broadcast_to(
        e_score_correction_bias[:, None], (N_EXP, L)
    ).astype(jnp.float32).reshape(N_EXP * L)
    mesh = plsc.VectorSubcoreMesh(
        core_axis_name="c", subcore_axis_name="s", num_cores=NC, num_subcores=NS
    )
    f = pl.kernel(
        _tec_body,
        out_type=(
            jax.ShapeDtypeStruct((N_TOK * TOPK,), jnp.int32),
            jax.ShapeDtypeStruct((N_TOK * TOPK,), jnp.float32),
        ),
        mesh=mesh,
        compiler_params=pltpu.CompilerParams(needs_layout_passes=False),
        scratch_types=[
            pltpu.VMEM((TPW * N_EXP,), jnp.float32),
            pltpu.VMEM((N_EXP * L,), jnp.float32),
            pltpu.VMEM((N_EXP * L,), jnp.float32),
            pltpu.VMEM((TPW * TOPK,), jnp.int32),
            pltpu.VMEM((TPW * TOPK,), jnp.float32),
        ],
    )
    oi, ow = f(logits_flat, bias_b)
    return oi.reshape(N_TOK, TOPK), ow.reshape(N_TOK, TOPK)


# speedup vs baseline: 2.6522x; 1.1429x over previous
import jax
import jax.numpy as jnp
from jax import lax
from jax.experimental import pallas as pl
from jax.experimental.pallas import tpu as pltpu
from jax.experimental.pallas import tpu_sc as plsc

N_TOK = 32768
N_EXP = 64
N_GRP = 8
GRP_SZ = 8
TOPK_GRP = 4
TOPK = 8
SCALE = 2.5

NC = 2
NS = 16
NW = NC * NS
TPW = N_TOK // NW
L = 16
TILES = TPW // L


def _i32(v):
    return jnp.full((L,), v, dtype=jnp.int32)


def _merge_top2(m1, s1, m2, s2):
    return (
        jnp.maximum(m1, m2),
        jnp.maximum(jnp.minimum(m1, m2), jnp.maximum(s1, s2)),
    )


def _tec_body(
    logits_hbm, bias_hbm, oi_hbm, ow_hbm, xs, s_buf, sf_buf, bias_v, oi_v, ow_v
):
    wid = lax.axis_index("s") * NC + lax.axis_index("c")
    base = wid * TPW
    pltpu.sync_copy(logits_hbm.at[pl.ds(base * N_EXP, TPW * N_EXP)], xs)
    pltpu.sync_copy(bias_hbm, bias_v)

    lanes = lax.iota(jnp.int32, L)
    neg_inf = jnp.full((L,), -jnp.inf, dtype=jnp.float32)
    NCAND = TOPK_GRP * GRP_SZ

    def tile(t, carry):
        tok_vec = t * L + lanes
        xbase = tok_vec * N_EXP

        sf = []
        for e in range(N_EXP):
            xe = plsc.load_gather(xs, [xbase + e])
            se = 1.0 / (1.0 + jnp.exp(-xe))
            s_buf[pl.ds(e * L, L)] = se
            sfe = se + bias_v[pl.ds(e * L, L)]
            sf_buf[pl.ds(e * L, L)] = sfe
            sf.append(sfe)

        gs = []
        for g in range(N_GRP):
            v = sf[GRP_SZ * g : GRP_SZ * (g + 1)]
            pm = [jnp.maximum(v[2 * i], v[2 * i + 1]) for i in range(4)]
            ps = [jnp.minimum(v[2 * i], v[2 * i + 1]) for i in range(4)]
            m01, s01 = _merge_top2(pm[0], ps[0], pm[1], ps[1])
            m23, s23 = _merge_top2(pm[2], ps[2], pm[3], ps[3])
            m, sec = _merge_top2(m01, s01, m23, s23)
            gs.append(m + sec)

        gsel = []
        for g in range(N_GRP):
            r = jnp.zeros((L,), dtype=jnp.int32)
            for h in range(N_GRP):
                if h == g:
                    continue
                c = (gs[h] >= gs[g]) if h < g else (gs[h] > gs[g])
                r = r + c.astype(jnp.int32)
            gsel.append(r < TOPK_GRP)

        sg = [_i32(0) for _ in range(TOPK_GRP)]
        cnt = jnp.zeros((L,), dtype=jnp.int32)
        for g in range(N_GRP):
            for r in range(TOPK_GRP):
                hit = gsel[g] & (cnt == r)
                sg[r] = jnp.where(hit, _i32(g), sg[r])
            cnt = cnt + gsel[g].astype(jnp.int32)

        eid = []
        cand = []
        for j in range(NCAND):
            e_j = (sg[j // GRP_SZ] << 3) + (j % GRP_SZ)
            eid.append(e_j)
            cand.append(plsc.load_gather(sf_buf, [e_j * L + lanes]))

        obase = tok_vec * TOPK
        ws = []
        for k in range(TOPK):
            m = cand[0]
            for j in range(1, NCAND):
                m = jnp.maximum(m, cand[j])
            idx = _i32(N_EXP)
            for j in range(NCAND):
                idx = jnp.minimum(idx, jnp.where(cand[j] == m, eid[j], _i32(N_EXP)))
            plsc.store_scatter(oi_v, [obase + k], idx)
            ws.append(plsc.load_gather(s_buf, [idx * L + lanes]))
            cand = [
                jnp.where(eid[j] == idx, neg_inf, cand[j]) for j in range(NCAND)
            ]

        den = ws[0]
        for k in range(1, TOPK):
            den = den + ws[k]
        inv = SCALE / (den + 1e-20)
        for k in range(TOPK):
            plsc.store_scatter(ow_v, [obase + k], ws[k] * inv)
        return carry

    lax.fori_loop(0, TILES, tile, 0)
    pltpu.sync_copy(oi_v, oi_hbm.at[pl.ds(base * TOPK, TPW * TOPK)])
    pltpu.sync_copy(ow_v, ow_hbm.at[pl.ds(base * TOPK, TPW * TOPK)])


@jax.jit
def kernel(router_logits, e_score_correction_bias):
    logits_flat = router_logits.reshape(N_TOK * N_EXP)
    bias_b = jnp.broadcast_to(
        e_score_correction_bias[:, None], (N_EXP, L)
    ).astype(jnp.float32).reshape(N_EXP * L)
    mesh = plsc.VectorSubcoreMesh(
        core_axis_name="c", subcore_axis_name="s", num_cores=NC, num_subcores=NS
    )
    f = pl.kernel(
        _tec_body,
        out_type=(
            jax.ShapeDtypeStruct((N_TOK * TOPK,), jnp.int32),
            jax.ShapeDtypeStruct((N_TOK * TOPK,), jnp.float32),
        ),
        mesh=mesh,
        compiler_params=pltpu.CompilerParams(needs_layout_passes=False),
        scratch_types=[
            pltpu.VMEM((TPW * N_EXP,), jnp.float32),
            pltpu.VMEM((N_EXP * L,), jnp.float32),
            pltpu.VMEM((N_EXP * L,), jnp.float32),
            pltpu.VMEM((N_EXP * L,), jnp.float32),
            pltpu.VMEM((TPW * TOPK,), jnp.int32),
            pltpu.VMEM((TPW * TOPK,), jnp.float32),
        ],
    )
    oi, ow = f(logits_flat, bias_b)
    return oi.reshape(N_TOK, TOPK), ow.reshape(N_TOK, TOPK)


# speedup vs baseline: 2.7348x; 1.0311x over previous
import jax
import jax.numpy as jnp
from jax import lax
from jax.experimental import pallas as pl
from jax.experimental.pallas import tpu as pltpu
from jax.experimental.pallas import tpu_sc as plsc

N_TOK = 32768
N_EXP = 64
N_GRP = 8
GRP_SZ = 8
TOPK_GRP = 4
TOPK = 8
SCALE = 2.5

NC = 2
NS = 16
NW = NC * NS
TPW = N_TOK // NW
L = 16
TILES = TPW // L


def _i32(v):
    return jnp.full((L,), v, dtype=jnp.int32)


def _merge_top2(m1, s1, m2, s2):
    return (
        jnp.maximum(m1, m2),
        jnp.maximum(jnp.minimum(m1, m2), jnp.maximum(s1, s2)),
    )


def _tec_body(
    logits_hbm, bias_hbm, oi_hbm, ow_hbm, xs, xp, s_buf, sf_buf, bias_v, oi_v, ow_v
):
    wid = lax.axis_index("s") * NC + lax.axis_index("c")
    base = wid * TPW
    pltpu.sync_copy(logits_hbm.at[pl.ds(base * N_EXP, TPW * N_EXP)], xs)
    pltpu.sync_copy(bias_hbm, bias_v)

    lanes = lax.iota(jnp.int32, L)
    neg_inf = jnp.full((L,), -jnp.inf, dtype=jnp.float32)
    NCAND = TOPK_GRP * GRP_SZ

    STRIDE = N_EXP + 1
    lanes_p = lanes * STRIDE

    def tile(t, carry):
        tok_vec = t * L + lanes

        for r in range(L):
            row = (t * L + r) * N_EXP
            for q in range(4):
                xp[pl.ds(r * STRIDE + q * L, L)] = xs[pl.ds(row + q * L, L)]

        sf = []
        for e in range(N_EXP):
            xe = plsc.load_gather(xp, [lanes_p + e])
            se = 1.0 / (1.0 + jnp.exp(-xe))
            s_buf[pl.ds(e * L, L)] = se
            sfe = se + bias_v[pl.ds(e * L, L)]
            sf_buf[pl.ds(e * L, L)] = sfe
            sf.append(sfe)

        gs = []
        for g in range(N_GRP):
            v = sf[GRP_SZ * g : GRP_SZ * (g + 1)]
            pm = [jnp.maximum(v[2 * i], v[2 * i + 1]) for i in range(4)]
            ps = [jnp.minimum(v[2 * i], v[2 * i + 1]) for i in range(4)]
            m01, s01 = _merge_top2(pm[0], ps[0], pm[1], ps[1])
            m23, s23 = _merge_top2(pm[2], ps[2], pm[3], ps[3])
            m, sec = _merge_top2(m01, s01, m23, s23)
            gs.append(m + sec)

        gsel = []
        for g in range(N_GRP):
            r = jnp.zeros((L,), dtype=jnp.int32)
            for h in range(N_GRP):
                if h == g:
                    continue
                c = (gs[h] >= gs[g]) if h < g else (gs[h] > gs[g])
                r = r + c.astype(jnp.int32)
            gsel.append(r < TOPK_GRP)

        sg = [_i32(0) for _ in range(TOPK_GRP)]
        cnt = jnp.zeros((L,), dtype=jnp.int32)
        for g in range(N_GRP):
            for r in range(TOPK_GRP):
                hit = gsel[g] & (cnt == r)
                sg[r] = jnp.where(hit, _i32(g), sg[r])
            cnt = cnt + gsel[g].astype(jnp.int32)

        eid = []
        cand = []
        for j in range(NCAND):
            e_j = (sg[j // GRP_SZ] << 3) + (j % GRP_SZ)
            eid.append(e_j)
            cand.append(plsc.load_gather(sf_buf, [e_j * L + lanes]))

        obase = tok_vec * TOPK
        ws = []
        for k in range(TOPK):
            m = cand[0]
            for j in range(1, NCAND):
                m = jnp.maximum(m, cand[j])
            idx = _i32(N_EXP)
            for j in range(NCAND):
                idx = jnp.minimum(idx, jnp.where(cand[j] == m, eid[j], _i32(N_EXP)))
            plsc.store_scatter(oi_v, [obase + k], idx)
            ws.append(plsc.load_gather(s_buf, [idx * L + lanes]))
            cand = [
                jnp.where(eid[j] == idx, neg_inf, cand[j]) for j in range(NCAND)
            ]

        den = ws[0]
        for k in range(1, TOPK):
            den = den + ws[k]
        inv = SCALE / (den + 1e-20)
        for k in range(TOPK):
            plsc.store_scatter(ow_v, [obase + k], ws[k] * inv)
        return carry

    lax.fori_loop(0, TILES, tile, 0)
    pltpu.sync_copy(oi_v, oi_hbm.at[pl.ds(base * TOPK, TPW * TOPK)])
    pltpu.sync_copy(ow_v, ow_hbm.at[pl.ds(base * TOPK, TPW * TOPK)])


@jax.jit
def kernel(router_logits, e_score_correction_bias):
    logits_flat = router_logits.reshape(N_TOK * N_EXP)
    bias_b = jnp.broadcast_to(
        e_score_correction_bias[:, None], (N_EXP, L)
    ).astype(jnp.float32).reshape(N_EXP * L)
    mesh = plsc.VectorSubcoreMesh(
        core_axis_name="c", subcore_axis_name="s", num_cores=NC, num_subcores=NS
    )
    f = pl.kernel(
        _tec_body,
        out_type=(
            jax.ShapeDtypeStruct((N_TOK * TOPK,), jnp.int32),
            jax.ShapeDtypeStruct((N_TOK * TOPK,), jnp.float32),
        ),
        mesh=mesh,
        compiler_params=pltpu.CompilerParams(needs_layout_passes=False),
        scratch_types=[
            pltpu.VMEM((TPW * N_EXP,), jnp.float32),
            pltpu.VMEM((L * (N_EXP + 1),), jnp.float32),
            pltpu.VMEM((N_EXP * L,), jnp.float32),
            pltpu.VMEM((N_EXP * L,), jnp.float32),
            pltpu.VMEM((N_EXP * L,), jnp.float32),
            pltpu.VMEM((TPW * TOPK,), jnp.int32),
            pltpu.VMEM((TPW * TOPK,), jnp.float32),
        ],
    )
    oi, ow = f(logits_flat, bias_b)
    return oi.reshape(N_TOK, TOPK), ow.reshape(N_TOK, TOPK)
